# Initial kernel scaffold; baseline (speedup 1.0000x reference)
import jax
import jax.numpy as jnp
from jax.experimental import pallas as pl


def kernel(carts, species, senders, receivers, incoming, incoming_pad, outgoing, outgoing_pad, node_graph_i, edge_graph_i, params):
    raise NotImplementedError("write your pallas kernel here")


# speedup vs baseline: 2.1835x; 2.1835x over previous
import functools
import numpy as np
import jax
import jax.numpy as jnp
from jax import lax
from jax.experimental import pallas as pl

_NR = 8
_NS = 4
_CUTOFF = 5.0
_MAXIN = 8
_MAXOUT = 8
_NG = 16
_BE = 2000
_BN = 250

_F32 = jnp.float32
_BF16 = jnp.bfloat16


def _mlp(x, w1, b1, w2, b2):
    return jax.nn.sigmoid(jnp.dot(x, w1, preferred_element_type=_F32) + b1) @ w2 + b2


def _full(shape):
    return pl.BlockSpec(shape, lambda i: tuple(0 for _ in shape))


def _blk(shape):
    return pl.BlockSpec(shape, lambda i: (i,) + tuple(0 for _ in shape[1:]))


def _e0_body(sp, rp, zi, zj, wdp, eproj, w1e, w1i, w1j, b1, w2, b2,
             wdo, dew1, deb1, dew2, deb2, wmsg, bmsg,
             o_demb, o_unit, o_msgs, o_em, o_zr, o_mp):
    d4 = rp[...] - sp[...]
    d2 = jnp.sum(d4 * d4, axis=1, keepdims=True) + 1e-12
    dist = jnp.sqrt(d2)
    nvec = (lax.broadcasted_iota(jnp.int32, (1, _NR), 1) + 1).astype(_F32)
    rbf = np.sqrt(2.0 / _CUTOFF) * jnp.sin(nvec * np.pi * dist / _CUTOFF) / (dist + 1e-8)
    demb = jnp.dot(rbf, wdp[...], preferred_element_type=_F32)
    unit = d4 / (dist + 1e-8)
    lane4 = lax.broadcasted_iota(jnp.int32, (d4.shape[0], 4), 1)
    o_unit[...] = jnp.where(lane4 == 3, dist, unit)
    o_demb[...] = demb
    eij = jnp.dot(demb, eproj[...], preferred_element_type=_F32)
    h = jax.nn.sigmoid(
        jnp.dot(eij, w1e[...], preferred_element_type=_F32)
        + jnp.dot(zi[...], w1i[...], preferred_element_type=_F32)
        + jnp.dot(zj[...], w1j[...], preferred_element_type=_F32)
        + b1[...])
    msgs = jnp.dot(h, w2[...], preferred_element_type=_F32) + b2[...]
    o_msgs[...] = msgs
    o_em[...] = msgs * jnp.dot(demb, wdo[...], preferred_element_type=_F32)
    o_zr[...] = _mlp(demb, dew1[...], deb1[...], dew2[...], deb2[...]).astype(_BF16)
    o_mp[...] = jax.nn.sigmoid(msgs @ wmsg[...] + bmsg[...]).astype(_BF16)


def _call_e0(E, sp, rp, zi, zj, p):
    ie = p['init_embed']
    w1 = ie['W1']
    de = p['int0']['dist_enc']
    outs = (
        jax.ShapeDtypeStruct((E, 128), _F32),
        jax.ShapeDtypeStruct((E, 4), _F32),
        jax.ShapeDtypeStruct((E, 64), _F32),
        jax.ShapeDtypeStruct((E, 64), _F32),
        jax.ShapeDtypeStruct((E, 64), _BF16),
        jax.ShapeDtypeStruct((E, 64), _BF16),
    )
    grid = E // _BE
    return pl.pallas_call(
        _e0_body,
        grid=(grid,),
        in_specs=[
            _blk((_BE, 4)), _blk((_BE, 4)), _blk((_BE, 128)), _blk((_BE, 128)),
            _full((8, 128)), _full((128, 64)),
            _full((64, 64)), _full((128, 64)), _full((128, 64)),
            _full((1, 64)), _full((64, 64)), _full((1, 64)),
            _full((128, 64)),
            _full((128, 64)), _full((1, 64)), _full((64, 64)), _full((1, 64)),
            _full((64, 64)), _full((1, 64)),
        ],
        out_specs=[
            _blk((_BE, 128)), _blk((_BE, 4)), _blk((_BE, 64)),
            _blk((_BE, 64)), _blk((_BE, 64)), _blk((_BE, 64)),
        ],
        out_shape=outs,
    )(sp, rp, zi, zj,
      p['W_distproj'], p['edge_proj'],
      w1[:64], w1[64:192], w1[192:], ie['b1'][None, :], ie['W2'], ie['b2'][None, :],
      p['out0']['W_distout'],
      de['W1'], de['b1'][None, :], de['W2'], de['b2'][None, :],
      p['int0']['Wmsg'], p['int0']['bmsg'][None, :])


_RM = np.zeros((_NR, _NR * _NS), np.float32)
for _r in range(_NR):
    _RM[_r, _r * _NS:(_r + 1) * _NS] = 1.0
_SM = np.zeros((_NS, _NR * _NS), np.float32)
for _s in range(_NS):
    _SM[_s, np.arange(_NR) * _NS + _s] = 1.0


def _trip_body(vin, vout, rm, sm, o_a):
    bn = vin.shape[0]
    vi = vin[...]
    vo = vout[...]
    R = bn * _MAXIN * _MAXOUT
    vi_t = jnp.reshape(
        jnp.broadcast_to(vi[:, :, None, :], (bn, _MAXIN, _MAXOUT, 4)), (R, 4))
    vo_t = jnp.reshape(
        jnp.broadcast_to(vo[:, None, :, :], (bn, _MAXIN, _MAXOUT, 4)), (R, 4))
    lane = lax.broadcasted_iota(jnp.int32, (R, 4), 1)
    vi3 = jnp.where(lane == 3, 0.0, vi_t)
    dots = jnp.sum(vi3 * vo_t, axis=-1, keepdims=True)
    xf = jnp.clip(-dots, -1.0 + 1e-7, 1.0 - 1e-7)
    df = jnp.sum(jnp.where(lane == 3, vi_t, 0.0), axis=-1, keepdims=True)
    nvec = (lax.broadcasted_iota(jnp.int32, (1, _NR), 1) + 1).astype(_F32)
    rbf = np.sqrt(2.0 / _CUTOFF) * jnp.sin(nvec * np.pi * df / _CUTOFF) / (df + 1e-8)
    A = jnp.dot(rbf, rm[...], preferred_element_type=_F32)
    t2 = 2.0 * xf * xf - 1.0
    t3 = (4.0 * xf * xf - 3.0) * xf
    smv = sm[...]
    t = smv[0:1, :] + xf * smv[1:2, :] + t2 * smv[2:3, :] + t3 * smv[3:4, :]
    o_a[...] = (A * t).astype(_BF16)


def _call_trip(N, vin, vout):
    grid = N // _BN
    R = _BN * _MAXIN * _MAXOUT
    return pl.pallas_call(
        _trip_body,
        grid=(grid,),
        in_specs=[_blk((_BN, _MAXIN, 4)), _blk((_BN, _MAXOUT, 4)),
                  _full((_NR, _NR * _NS)), _full((_NS, _NR * _NS))],
        out_specs=_blk((R, _NR * _NS)),
        out_shape=jax.ShapeDtypeStruct((N * _MAXIN * _MAXOUT, _NR * _NS), _BF16),
    )(vin, vout, jnp.asarray(_RM), jnp.asarray(_SM))


def _int_body(zji, mkj, a, dw1, db1, dw2, db2, aw1, ab1, aw2, ab2, wup, bup, o_ma):
    bn = zji.shape[0] // _MAXOUT
    R = bn * _MAXIN * _MAXOUT
    z3 = jnp.reshape(zji[...].astype(_F32), (bn, _MAXOUT, 64))
    m3 = jnp.reshape(mkj[...].astype(_F32), (bn, _MAXIN, 64))
    md = (m3[:, :, None, :] * z3[:, None, :, :]).astype(_BF16).astype(_F32)
    mdf = jnp.reshape(md, (R, 64))
    mdown = _mlp(mdf, dw1[...], db1[...], dw2[...], db2[...])
    a32 = a[...].astype(_F32)
    za = _mlp(a32, aw1[...], ab1[...], aw2[...], ab2[...])
    prod = jnp.reshape(mdown * za, (bn, _MAXIN, _MAXOUT, 32))
    s = jnp.reshape(jnp.sum(prod, axis=1), (bn * _MAXOUT, 32))
    o_ma[...] = jax.nn.sigmoid(jnp.dot(s, wup[...], preferred_element_type=_F32) + bup[...])


def _call_int(N, zji_f, mkj_f, a_f, pi):
    grid = N // _BN
    R = _BN * _MAXIN * _MAXOUT
    dn, an = pi['down'], pi['ang']
    return pl.pallas_call(
        _int_body,
        grid=(grid,),
        in_specs=[
            _blk((_BN * _MAXOUT, 64)), _blk((_BN * _MAXIN, 64)), _blk((R, 32)),
            _full((64, 32)), _full((1, 32)), _full((32, 32)), _full((1, 32)),
            _full((32, 32)), _full((1, 32)), _full((32, 32)), _full((1, 32)),
            _full((32, 64)), _full((1, 64)),
        ],
        out_specs=_blk((_BN * _MAXOUT, 64)),
        out_shape=jax.ShapeDtypeStruct((N * _MAXOUT, 64), _F32),
    )(zji_f, mkj_f, a_f,
      dn['W1'], dn['b1'][None, :], dn['W2'], dn['b2'][None, :],
      an['W1'], an['b1'][None, :], an['W2'], an['b2'][None, :],
      pi['Wup'], pi['bup'][None, :])


def _eu_body(has_next, msgs, segm, demb,
             wprev, bprev, pw1, pb1, pw2, pb2, qw1, qb1, qw2, qb2, wdo,
             *rest):
    if has_next:
        (dew1, deb1, dew2, deb2, wmsg, bmsg,
         o_msgs, o_em, o_zr, o_mp) = rest
    else:
        (o_msgs, o_em) = rest
    m = msgs[...]
    prev = jax.nn.sigmoid(m @ wprev[...] + bprev[...])
    combo = prev + segm[...]
    pre = _mlp(combo, pw1[...], pb1[...], pw2[...], pb2[...]) + m
    new = _mlp(pre, qw1[...], qb1[...], qw2[...], qb2[...])
    o_msgs[...] = new
    o_em[...] = new * jnp.dot(demb[...], wdo[...], preferred_element_type=_F32)
    if has_next:
        o_zr[...] = _mlp(demb[...], dew1[...], deb1[...], dew2[...], deb2[...]).astype(_BF16)
        o_mp[...] = jax.nn.sigmoid(new @ wmsg[...] + bmsg[...]).astype(_BF16)


def _call_eu(E, msgs, segm, demb, pi, wdo, pnext):
    has_next = pnext is not None
    grid = E // _BE
    in_specs = [
        _blk((_BE, 64)), _blk((_BE, 64)), _blk((_BE, 128)),
        _full((64, 64)), _full((1, 64)),
        _full((64, 64)), _full((1, 64)), _full((64, 64)), _full((1, 64)),
        _full((64, 64)), _full((1, 64)), _full((64, 64)), _full((1, 64)),
        _full((128, 64)),
    ]
    args = [msgs, segm, demb,
            pi['Wprev'], pi['bprev'][None, :],
            pi['pre']['W1'], pi['pre']['b1'][None, :], pi['pre']['W2'], pi['pre']['b2'][None, :],
            pi['post']['W1'], pi['post']['b1'][None, :], pi['post']['W2'], pi['post']['b2'][None, :],
            wdo]
    outs = [jax.ShapeDtypeStruct((E, 64), _F32), jax.ShapeDtypeStruct((E, 64), _F32)]
    out_specs = [_blk((_BE, 64)), _blk((_BE, 64))]
    if has_next:
        de = pnext['dist_enc']
        in_specs += [_full((128, 64)), _full((1, 64)), _full((64, 64)), _full((1, 64)),
                     _full((64, 64)), _full((1, 64))]
        args += [de['W1'], de['b1'][None, :], de['W2'], de['b2'][None, :],
                 pnext['Wmsg'], pnext['bmsg'][None, :]]
        outs += [jax.ShapeDtypeStruct((E, 64), _BF16), jax.ShapeDtypeStruct((E, 64), _BF16)]
        out_specs += [_blk((_BE, 64)), _blk((_BE, 64))]
    return pl.pallas_call(
        functools.partial(_eu_body, has_next),
        grid=(grid,),
        in_specs=in_specs,
        out_specs=out_specs,
        out_shape=tuple(outs),
    )(*args)


def _node_body(nm0, nm1, nm2, g, h0w1, h0b1, h0w2, h0b2,
               h1w1, h1b1, h1w2, h1b2, h2w1, h2b1, h2w2, h2b2, o):
    N = nm0.shape[0]
    gi = g[...]
    oh = (lax.broadcasted_iota(jnp.int32, (N, _NG), 1) == gi).astype(_F32)
    cnt = jnp.sum(oh, axis=0)[:, None]
    dn = (((0,), (0,)), ((), ()))
    acc = jnp.zeros((_NG, 1), _F32)
    for nm, w1, b1, w2, b2 in ((nm0, h0w1, h0b1, h0w2, h0b2),
                               (nm1, h1w1, h1b1, h1w2, h1b2),
                               (nm2, h2w1, h2b1, h2w2, h2b2)):
        no = _mlp(nm[...], w1[...], b1[...], w2[...], b2[...])
        acc = acc + lax.dot_general(oh, no, dn, preferred_element_type=_F32)
    o[...] = acc / (1e-6 + cnt)


def _call_node(N, nm0, nm1, nm2, g2, heads):
    args = [nm0, nm1, nm2, g2]
    in_specs = [_full((N, 64)), _full((N, 64)), _full((N, 64)), _full((N, 1))]
    for h in heads:
        args += [h['W1'], h['b1'][None, :], h['W2'], h['b2'][None, :]]
        in_specs += [_full((64, 64)), _full((1, 64)), _full((64, 1)), _full((1, 1))]
    return pl.pallas_call(
        _node_body,
        grid=(1,),
        in_specs=in_specs,
        out_specs=_full((_NG, 1)),
        out_shape=jax.ShapeDtypeStruct((_NG, 1), _F32),
    )(*args)


def kernel(carts, species, senders, receivers, incoming, incoming_pad,
           outgoing, outgoing_pad, node_graph_i, edge_graph_i, params):
    N = carts.shape[0]
    E = senders.shape[0]
    node_emb = jnp.take(params['emb_table'], species, axis=0)
    c4 = jnp.pad(carts, ((0, 0), (0, 1)))
    sp = jnp.take(c4, senders, axis=0)
    rp = jnp.take(c4, receivers, axis=0)
    zi = jnp.take(node_emb, senders, axis=0)
    zj = jnp.take(node_emb, receivers, axis=0)

    demb, unitd, msgs, em0, zr, mp = _call_e0(E, sp, rp, zi, zj, params)

    vin = jnp.take(unitd, incoming, axis=0)
    vout = jnp.take(unitd, outgoing, axis=0)
    a_f = _call_trip(N, vin, vout)

    out_flat = outgoing.reshape(-1)
    cnt_e = jax.ops.segment_sum(jnp.ones((N * _MAXOUT, 1), _F32), out_flat,
                                num_segments=E)
    cnt_r = jax.ops.segment_sum(jnp.ones((E, 1), _F32), receivers, num_segments=N)

    ems = [em0]
    for i in range(2):
        pi = params['int%d' % i]
        zji = jnp.take(zr, outgoing, axis=0).reshape(N * _MAXOUT, 64)
        mkj = jnp.take(mp, incoming, axis=0).reshape(N * _MAXIN, 64)
        ma = _call_int(N, zji, mkj, a_f, pi)
        seg = jax.ops.segment_sum(ma, out_flat, num_segments=E)
        segm = seg / (1e-6 + cnt_e)
        pnext = params['int%d' % (i + 1)] if i == 0 else None
        res = _call_eu(E, msgs, segm, demb, pi,
                       params['out%d' % (i + 1)]['W_distout'], pnext)
        if pnext is not None:
            msgs, emi, zr, mp = res
        else:
            msgs, emi = res
        ems.append(emi)

    nms = [jax.ops.segment_sum(e, receivers, num_segments=N) / (1e-6 + cnt_r)
           for e in ems]
    heads = [params['out%d' % j]['head'] for j in range(3)]
    g2 = node_graph_i.astype(jnp.int32)[:, None]
    return _call_node(N, nms[0], nms[1], nms[2], g2, heads)
